# baseline (speedup 1.0000x reference)
import functools

import jax
import jax.numpy as jnp
from jax.experimental import pallas as pl
from jax.experimental.pallas import tpu as pltpu


def _make_taps(H, W):
    L = H * W
    lane = jax.lax.broadcasted_iota(jnp.int32, (1, L), 1)
    row = lane // W
    col = lane % W
    taps = []
    for t in range(9):
        oy, ox = t // 3 - 1, t % 3 - 1
        shift = (-(oy * W + ox)) % L
        valid = ((row + oy >= 0) & (row + oy < H) &
                 (col + ox >= 0) & (col + ox < W))
        taps.append((shift, valid.astype(jnp.float32)))
    return taps


def _conv3x3(x, w_stacked, b, taps, relu=False):
    cout = w_stacked.shape[0] // 9
    y = jnp.dot(w_stacked, x, preferred_element_type=jnp.float32)
    acc = None
    for t, (shift, mask) in enumerate(taps):
        contrib = y[t * cout:(t + 1) * cout, :]
        if shift:
            contrib = pltpu.roll(contrib, shift, axis=1)
        if t != 4:
            contrib = contrib * mask
        acc = contrib if acc is None else acc + contrib
    acc = acc + b
    if relu:
        acc = jnp.maximum(acc, 0.0)
    return acc


def _main_kernel(x_ref, w1_ref, b1_ref, w2_ref, b2_ref,
                 wc_ref, bc_ref, wl_ref, bl_ref, wg0_ref,
                 bg0_ref, wg1_ref, bg1_ref, wu_ref, bu_ref,
                 out_ref, up_ref,
                 buf_ref, carry_ref, f1_ref, gacc_ref,
                 *, H, W, G0, G, C, D):
    taps = _make_taps(H, W)
    d = pl.program_id(1)

    @pl.when(d == 0)
    def _():
        f1 = _conv3x3(x_ref[0], w1_ref[...], b1_ref[...], taps)
        x2 = _conv3x3(f1.astype(jnp.bfloat16), w2_ref[...], b2_ref[...], taps)
        f1_ref[...] = f1
        carry_ref[...] = x2
        buf_ref[0:G0, :] = x2.astype(jnp.bfloat16)
        gacc_ref[...] = jnp.zeros_like(gacc_ref)

    for c in range(C):
        cin = G0 + c * G
        y = _conv3x3(buf_ref[0:cin, :], wc_ref[0, c, :, 0:cin], bc_ref[0, c],
                     taps, relu=True)
        buf_ref[cin:cin + G, :] = y.astype(jnp.bfloat16)

    lff = jnp.dot(wl_ref[0], buf_ref[0:G0 + C * G, :],
                  preferred_element_type=jnp.float32) + bl_ref[0]
    new_x = lff + carry_ref[...]
    carry_ref[...] = new_x
    nx16 = new_x.astype(jnp.bfloat16)
    buf_ref[0:G0, :] = nx16
    out_ref[0, 0] = new_x

    gacc_ref[...] += jnp.dot(wg0_ref[0], nx16, preferred_element_type=jnp.float32)

    @pl.when(d == D - 1)
    def _():
        y = gacc_ref[...] + bg0_ref[...]
        y = _conv3x3(y.astype(jnp.bfloat16), wg1_ref[...], bg1_ref[...], taps)
        y = y + f1_ref[...]
        up_ref[0] = _conv3x3(y.astype(jnp.bfloat16), wu_ref[...], bu_ref[...],
                             taps)


def _tail_kernel(x_ref, w_ref, b_ref, o_ref, *, H, W, n_colors):
    taps = _make_taps(H, W)
    y = _conv3x3(x_ref[0], w_ref[...], b_ref[...], taps)
    o_ref[0] = y[0:n_colors, :]


def _full(a):
    shape = a.shape
    return pl.BlockSpec(shape, lambda *_: (0,) * len(shape))


def _run_main(x, p, H, W, G0, G, C, D, up_cout):
    N, Cin, L = x.shape
    wc, bc, wl, bl, wg0 = (p["rdb_wc"], p["rdb_bc"], p["rdb_wl"], p["rdb_bl"],
                           p["gff0_w_d"])
    return pl.pallas_call(
        functools.partial(_main_kernel, H=H, W=W, G0=G0, G=G, C=C, D=D),
        out_shape=(jax.ShapeDtypeStruct((N, D, G0, L), jnp.float32),
                   jax.ShapeDtypeStruct((N, up_cout, L), jnp.float32)),
        grid=(N, D),
        in_specs=[
            pl.BlockSpec((1, Cin, L), lambda n, d: (n, 0, 0)),
            _full(p["sfe1_w"]), _full(p["sfe1_b"]),
            _full(p["sfe2_w"]), _full(p["sfe2_b"]),
            pl.BlockSpec((1,) + wc.shape[1:], lambda n, d: (d, 0, 0, 0)),
            pl.BlockSpec((1,) + bc.shape[1:], lambda n, d: (d, 0, 0, 0)),
            pl.BlockSpec((1,) + wl.shape[1:], lambda n, d: (d, 0, 0)),
            pl.BlockSpec((1,) + bl.shape[1:], lambda n, d: (d, 0, 0)),
            pl.BlockSpec((1,) + wg0.shape[1:], lambda n, d: (d, 0, 0)),
            _full(p["gff0_b"]),
            _full(p["gff1_w"]), _full(p["gff1_b"]),
            _full(p["up0_w"]), _full(p["up0_b"]),
        ],
        out_specs=(pl.BlockSpec((1, 1, G0, L), lambda n, d: (n, d, 0, 0)),
                   pl.BlockSpec((1, up_cout, L), lambda n, d: (n, 0, 0))),
        scratch_shapes=[pltpu.VMEM((G0 + C * G, L), jnp.bfloat16),
                        pltpu.VMEM((G0, L), jnp.float32),
                        pltpu.VMEM((G0, L), jnp.float32),
                        pltpu.VMEM((G0, L), jnp.float32)],
        compiler_params=pltpu.CompilerParams(
            dimension_semantics=("parallel", "arbitrary")),
    )(x, p["sfe1_w"], p["sfe1_b"], p["sfe2_w"], p["sfe2_b"],
      wc, bc, wl, bl, wg0, p["gff0_b"], p["gff1_w"], p["gff1_b"],
      p["up0_w"], p["up0_b"])


def _run_tail(xs, p, H, W, n_colors):
    N, Cin, L = xs.shape
    return pl.pallas_call(
        functools.partial(_tail_kernel, H=H, W=W, n_colors=n_colors),
        out_shape=jax.ShapeDtypeStruct((N, n_colors, L), jnp.float32),
        grid=(N,),
        in_specs=[pl.BlockSpec((1, Cin, L), lambda n: (n, 0, 0)),
                  _full(p["up2_w"]), _full(p["up2_b"])],
        out_specs=pl.BlockSpec((1, n_colors, L), lambda n: (n, 0, 0)),
        compiler_params=pltpu.CompilerParams(dimension_semantics=("parallel",)),
    )(xs, p["up2_w"], p["up2_b"])


def kernel(x, sfe1_w, sfe1_b, sfe2_w, sfe2_b, rdb_wc, rdb_bc, rdb_wl, rdb_bl,
           gff0_w_d, gff0_b, gff1_w, gff1_b, up0_w, up0_b, up2_w, up2_b):
    D, C, G, r, G0 = 20, 6, 32, 2, 8
    n_colors = 3
    bf = jnp.bfloat16
    p = {
        "sfe1_w": sfe1_w.astype(bf), "sfe1_b": sfe1_b,
        "sfe2_w": sfe2_w.astype(bf), "sfe2_b": sfe2_b,
        "rdb_wc": rdb_wc.astype(bf), "rdb_bc": rdb_bc,
        "rdb_wl": rdb_wl.astype(bf), "rdb_bl": rdb_bl,
        "gff0_w_d": gff0_w_d.astype(bf), "gff0_b": gff0_b,
        "gff1_w": gff1_w.astype(bf), "gff1_b": gff1_b,
        "up0_w": up0_w.astype(bf), "up0_b": up0_b,
        "up2_w": up2_w.astype(bf), "up2_b": up2_b,
    }
    N, _, H, W = x.shape

    xs = x.astype(bf).reshape(N, n_colors, H * W)
    bit = jnp.zeros((N,), jnp.float32)

    rdb_outs, up = _run_main(xs, p, H, W, G0, G, C, D, G * r * r)

    u = up.reshape(N, G, r, r, H, W)
    u = jnp.transpose(u, (0, 1, 4, 2, 5, 3)).reshape(N, G, H * r * W * r)
    out = _run_tail(u.astype(bf), p, H * r, W * r, n_colors)
    out = out.reshape(N, n_colors, H * r, W * r)

    norm = jnp.sqrt(jnp.sum(rdb_outs * rdb_outs, axis=(0, 2, 3)))
    feat = rdb_outs / norm[None, :, None, None] / float(G0 * H * W)
    feat = jnp.transpose(feat, (1, 0, 2, 3)).reshape(D * N, G0, H, W)

    return out, feat, bit


# speedup vs baseline: 1.6538x; 1.6538x over previous
import functools

import jax
import jax.numpy as jnp
from jax.experimental import pallas as pl
from jax.experimental.pallas import tpu as pltpu

BF = jnp.bfloat16
RW = 128


def _col_masks(ncols, width):
    lw = jax.lax.broadcasted_iota(jnp.int32, (1, ncols), 1) % width
    return lw != 0, lw != (width - 1)


def _variants(yb, mL, mR):
    zero = jnp.zeros_like(yb)
    n = yb.shape[1]
    yL = jnp.where(mL, pltpu.roll(yb, 1, axis=1), zero)
    yR = jnp.where(mR, pltpu.roll(yb, n - 1, axis=1), zero)
    return yL, yR


def _tapsum(dots, cout, b, L2, relu=False):
    acc = None
    for oy in (-1, 0, 1):
        for ox in (-1, 0, 1):
            blk = dots[ox + 1][(oy + 1) * cout:(oy + 2) * cout,
                               RW + 128 * oy: L2 - RW + 128 * oy]
            acc = blk if acc is None else acc + blk
    y = acc + b
    if relu:
        y = jnp.maximum(y, 0.0)
    return y


def _main_kernel(x_ref, w1_ref, b1_ref, w2_ref, b2_ref,
                 wc_ref, bc_ref, wl_ref, bl_ref, wg0_ref,
                 bg0_ref, wg1_ref, bg1_ref, wu_ref, bu_ref,
                 out_ref, up_ref,
                 b0_ref, bL_ref, bR_ref, carry_ref, f1_ref, gacc_ref, vb_ref,
                 *, H, G0, G, C, D):
    L2 = (H + 2) * 128
    win = slice(RW, L2 - RW)
    d = pl.program_id(1)
    mLw, mRw = _col_masks(L2 - 2 * RW, 64)
    mLs, mRs = _col_masks(L2, 64)

    def conv3(wall, k, bufs, cout, b, relu):
        dots = [jnp.dot(wall[v * 3 * cout:(v + 1) * 3 * cout, 0:k], bv,
                        preferred_element_type=jnp.float32)
                for v, bv in enumerate(bufs)]
        return _tapsum(dots, cout, b, L2, relu)

    def store_vb(yb):
        yL, yR = _variants(yb, mLw, mRw)
        vb_ref[0:G0, win] = yL
        vb_ref[16:16 + G0, win] = yb
        vb_ref[32:32 + G0, win] = yR

    @pl.when(d == 0)
    def _():
        for ref in (b0_ref, bL_ref, bR_ref):
            nrows = ref.shape[0]
            ref[:, 0:RW] = jnp.zeros((nrows, RW), BF)
            ref[:, L2 - RW:L2] = jnp.zeros((nrows, RW), BF)
        vb_ref[...] = jnp.zeros(vb_ref.shape, BF)
        gacc_ref[...] = jnp.zeros(gacc_ref.shape, jnp.float32)

        xs = x_ref[0]
        xL, xR = _variants(xs, mLs, mRs)
        f1 = conv3(w1_ref[...], 3, (xL, xs, xR), G0, b1_ref[...], False)
        f1_ref[...] = f1
        store_vb(f1.astype(BF))
        x2 = conv3(w2_ref[...], G0,
                   (vb_ref[0:G0, :], vb_ref[16:16 + G0, :], vb_ref[32:32 + G0, :]),
                   G0, b2_ref[...], False)
        carry_ref[...] = x2
        xb = x2.astype(BF)
        xbL, xbR = _variants(xb, mLw, mRw)
        b0_ref[0:G0, win] = xb
        bL_ref[0:G0, win] = xbL
        bR_ref[0:G0, win] = xbR

    for c in range(C):
        k = G0 + 32 * c
        y = conv3(wc_ref[0, c], k,
                  (bL_ref[0:k, :], b0_ref[0:k, :], bR_ref[0:k, :]),
                  G, bc_ref[0, c], True)
        yb = y.astype(BF)
        b0_ref[k:k + G, win] = yb
        if c < C - 1:
            yL, yR = _variants(yb, mLw, mRw)
            bL_ref[k:k + G, win] = yL
            bR_ref[k:k + G, win] = yR

    lff = jnp.dot(wl_ref[0], b0_ref[:, win],
                  preferred_element_type=jnp.float32) + bl_ref[0]
    new_x = lff + carry_ref[...]
    carry_ref[...] = new_x
    out_ref[0, 0] = new_x
    nx = new_x.astype(BF)
    nxL, nxR = _variants(nx, mLw, mRw)
    b0_ref[0:G0, win] = nx
    bL_ref[0:G0, win] = nxL
    bR_ref[0:G0, win] = nxR

    gacc_ref[...] += jnp.dot(wg0_ref[0], nx, preferred_element_type=jnp.float32)

    @pl.when(d == D - 1)
    def _():
        g = gacc_ref[...] + bg0_ref[...]
        store_vb(g.astype(BF))
        y = conv3(wg1_ref[...], G0,
                  (vb_ref[0:G0, :], vb_ref[16:16 + G0, :], vb_ref[32:32 + G0, :]),
                  G0, bg1_ref[...], False)
        y = y + f1_ref[...]
        store_vb(y.astype(BF))
        up = conv3(wu_ref[...], G0,
                   (vb_ref[0:G0, :], vb_ref[16:16 + G0, :], vb_ref[32:32 + G0, :]),
                   4 * G, bu_ref[...], False)
        up_ref[0] = up


def _tail_kernel(x_ref, w_ref, b_ref, o_ref, *, H, n_colors):
    L2 = (H + 2) * 128
    mLs, mRs = _col_masks(L2, 128)
    xs = x_ref[0]
    xL, xR = _variants(xs, mLs, mRs)
    dots = [jnp.dot(w_ref[v * 24:(v + 1) * 24, :], bv,
                    preferred_element_type=jnp.float32)
            for v, bv in enumerate((xL, xs, xR))]
    y = _tapsum(dots, 8, b_ref[...], L2)
    o_ref[0] = y[0:n_colors, :]


def _full(a):
    shape = a.shape
    return pl.BlockSpec(shape, lambda *_: (0,) * len(shape))


def _oxmajor(w, cout):
    return w.reshape(3, 3, cout, -1).transpose(1, 0, 2, 3).reshape(9 * cout, -1)


def kernel(x, sfe1_w, sfe1_b, sfe2_w, sfe2_b, rdb_wc, rdb_bc, rdb_wl, rdb_bl,
           gff0_w_d, gff0_b, gff1_w, gff1_b, up0_w, up0_b, up2_w, up2_b):
    D, C, G, r, G0 = 20, 6, 32, 2, 8
    n_colors = 3
    N, _, H, W = x.shape
    P = N // 2
    L2 = (H + 2) * 128
    Lw = H * 128

    w1 = _oxmajor(sfe1_w, G0).astype(BF)
    w2 = _oxmajor(sfe2_w, G0).astype(BF)
    wg1 = _oxmajor(gff1_w, G0).astype(BF)
    wu = _oxmajor(up0_w, 4 * G).astype(BF)
    wt = _oxmajor(up2_w, 8).astype(BF)

    wc2 = rdb_wc.reshape(D, C, 3, 3, G, G0 + (C - 1) * G)
    wc2 = wc2.transpose(0, 1, 3, 2, 4, 5).reshape(D, C, 9 * G, G0 + (C - 1) * G)
    wc2 = wc2.astype(BF)
    KB = G0 + (C - 1) * G
    KL = G0 + C * G
    wl2 = rdb_wl.astype(BF)
    wg0 = gff0_w_d.astype(BF)

    xi = x.astype(BF).reshape(P, 2, n_colors, H, W)
    xi = xi.transpose(0, 2, 3, 1, 4).reshape(P, n_colors, H * 128)
    xp = jnp.pad(xi, ((0, 0), (0, 0), (RW, RW)))

    grid = (P, D)
    rdb_outs, up = pl.pallas_call(
        functools.partial(_main_kernel, H=H, G0=G0, G=G, C=C, D=D),
        out_shape=(jax.ShapeDtypeStruct((P, D, G0, Lw), jnp.float32),
                   jax.ShapeDtypeStruct((P, 4 * G, Lw), jnp.float32)),
        grid=grid,
        in_specs=[
            pl.BlockSpec((1, n_colors, L2), lambda p, d: (p, 0, 0)),
            _full(w1), _full(sfe1_b), _full(w2), _full(sfe2_b),
            pl.BlockSpec((1,) + wc2.shape[1:], lambda p, d: (d, 0, 0, 0)),
            pl.BlockSpec((1,) + rdb_bc.shape[1:], lambda p, d: (d, 0, 0, 0)),
            pl.BlockSpec((1,) + wl2.shape[1:], lambda p, d: (d, 0, 0)),
            pl.BlockSpec((1,) + rdb_bl.shape[1:], lambda p, d: (d, 0, 0)),
            pl.BlockSpec((1,) + wg0.shape[1:], lambda p, d: (d, 0, 0)),
            _full(gff0_b), _full(wg1), _full(gff1_b), _full(wu), _full(up0_b),
        ],
        out_specs=(pl.BlockSpec((1, 1, G0, Lw), lambda p, d: (p, d, 0, 0)),
                   pl.BlockSpec((1, 4 * G, Lw), lambda p, d: (p, 0, 0))),
        scratch_shapes=[pltpu.VMEM((KL, L2), BF),
                        pltpu.VMEM((KB, L2), BF),
                        pltpu.VMEM((KB, L2), BF),
                        pltpu.VMEM((G0, Lw), jnp.float32),
                        pltpu.VMEM((G0, Lw), jnp.float32),
                        pltpu.VMEM((G0, Lw), jnp.float32),
                        pltpu.VMEM((48, L2), BF)],
        compiler_params=pltpu.CompilerParams(
            dimension_semantics=("parallel", "arbitrary")),
    )(xp, w1, sfe1_b, w2, sfe2_b, wc2, rdb_bc, wl2, rdb_bl, wg0,
      gff0_b, wg1, gff1_b, wu, up0_b)

    ro = rdb_outs.reshape(P, D, G0, H, 2, W).transpose(0, 4, 1, 2, 3, 5)
    ro = ro.reshape(N, D, G0, H * W)

    u0 = up.reshape(P, 4 * G, H, 2, W).transpose(0, 3, 1, 2, 4).reshape(N, 4 * G, H, W)
    u = u0.reshape(N, G, r, r, H, W).transpose(0, 1, 4, 2, 5, 3)
    u = u.reshape(N, G, H * r, W * r)
    ut = jnp.pad(u.astype(BF), ((0, 0), (0, 0), (1, 1), (0, 0)))
    ut = ut.reshape(N, G, (H * r + 2) * W * r)

    out = pl.pallas_call(
        functools.partial(_tail_kernel, H=H * r, n_colors=n_colors),
        out_shape=jax.ShapeDtypeStruct((N, n_colors, H * r * W * r), jnp.float32),
        grid=(N,),
        in_specs=[pl.BlockSpec((1, G, (H * r + 2) * W * r), lambda n: (n, 0, 0)),
                  _full(wt), _full(up2_b)],
        out_specs=pl.BlockSpec((1, n_colors, H * r * W * r), lambda n: (n, 0, 0)),
        compiler_params=pltpu.CompilerParams(dimension_semantics=("parallel",)),
    )(ut, wt, up2_b)
    out = out.reshape(N, n_colors, H * r, W * r)

    norm = jnp.sqrt(jnp.sum(ro * ro, axis=(0, 2, 3)))
    feat = ro / norm[None, :, None, None] / float(G0 * H * W)
    feat = feat.transpose(1, 0, 2, 3).reshape(D * N, G0, H, W)

    bit = jnp.zeros((N,), jnp.float32)
    return out, feat, bit
